# baseline (speedup 1.0000x reference)
import functools

import jax
import jax.numpy as jnp
from jax import lax
from jax.experimental import pallas as pl
from jax.experimental.pallas import tpu as pltpu
from jax.experimental.pallas import tpu_sc as plsc

NUM_CORES = 2
NUM_SUBCORES = 16
LANES = 16
NUM_WORKERS = NUM_CORES * NUM_SUBCORES

ROWS_PER_STREAM = 128
STREAMS_PER_ITEM = 8
ITEM = ROWS_PER_STREAM * STREAMS_PER_ITEM


def _make_gather(n_trees, n_leaves, batch, n_classes):
  total_rows = n_trees * batch
  assert total_rows % (NUM_WORKERS * ITEM) == 0
  n_items = total_rows // ITEM
  items_per_worker = n_items // NUM_WORKERS
  assert batch % ITEM == 0
  items_per_tree = batch // ITEM

  mesh = plsc.VectorSubcoreMesh(
      core_axis_name="c", subcore_axis_name="s",
      num_cores=NUM_CORES, num_subcores=NUM_SUBCORES)

  @functools.partial(
      pl.kernel,
      mesh=mesh,
      compiler_params=pltpu.CompilerParams(use_tc_tiling_on_sc=False),
      out_type=jax.ShapeDtypeStruct((total_rows, n_classes), jnp.float32),
      scratch_types=[
          pltpu.VMEM((STREAMS_PER_ITEM, ROWS_PER_STREAM), jnp.int32),
          pltpu.VMEM((ITEM, n_classes), jnp.float32),
          pltpu.SemaphoreType.DMA,
      ],
  )
  def gather_kernel(table, idx3, out, idx_v, rows_v, sem):
    wid = lax.axis_index("s") * NUM_CORES + lax.axis_index("c")

    @pl.loop(0, items_per_worker)
    def _item(j):
      m = wid * items_per_worker + j
      tree = m // items_per_tree
      offset = tree * n_leaves

      pltpu.sync_copy(idx3.at[m], idx_v)

      for r in range(STREAMS_PER_ITEM):
        for v in range(ROWS_PER_STREAM // LANES):
          sl = pl.ds(v * LANES, LANES)
          idx_v[r, sl] = idx_v[r, sl] + offset

      copies = []
      for r in range(STREAMS_PER_ITEM):
        dst = rows_v.at[pl.ds(r * ROWS_PER_STREAM, ROWS_PER_STREAM)]
        copies.append(pltpu.async_copy(table.at[idx_v.at[r]], dst, sem))
      for c in copies:
        c.wait()

      pltpu.sync_copy(rows_v, out.at[pl.ds(m * ITEM, ITEM)])

  return gather_kernel


def kernel(leafs, idx):
  n_trees, n_leaves, n_classes = leafs.shape
  batch = idx.shape[1]
  table = leafs.reshape(n_trees * n_leaves, n_classes)
  idx3 = idx.astype(jnp.int32).reshape(-1, STREAMS_PER_ITEM, ROWS_PER_STREAM)
  out = _make_gather(n_trees, n_leaves, batch, n_classes)(table, idx3)
  return out.reshape(n_trees, batch, n_classes)


# speedup vs baseline: 1.0079x; 1.0079x over previous
import functools

import jax
import jax.numpy as jnp
from jax import lax
from jax.experimental import pallas as pl
from jax.experimental.pallas import tpu as pltpu
from jax.experimental.pallas import tpu_sc as plsc

NUM_CORES = 2
NUM_SUBCORES = 16
LANES = 16
NUM_WORKERS = NUM_CORES * NUM_SUBCORES

ROWS_PER_STREAM = 128
STREAMS_PER_ITEM = 4
ITEM = ROWS_PER_STREAM * STREAMS_PER_ITEM
NBUF = 2


def _make_gather(n_trees, n_leaves, batch, n_classes):
  total_rows = n_trees * batch
  n_items = total_rows // ITEM
  items_per_worker = n_items // NUM_WORKERS
  assert n_items == items_per_worker * NUM_WORKERS
  assert batch % ITEM == 0
  items_per_tree = batch // ITEM
  assert items_per_worker % NBUF == 0
  n_groups = items_per_worker // NBUF

  mesh = plsc.VectorSubcoreMesh(
      core_axis_name="c", subcore_axis_name="s",
      num_cores=NUM_CORES, num_subcores=NUM_SUBCORES)

  @functools.partial(
      pl.kernel,
      mesh=mesh,
      compiler_params=pltpu.CompilerParams(use_tc_tiling_on_sc=False),
      out_type=jax.ShapeDtypeStruct((total_rows, n_classes), jnp.float32),
      scratch_types=[
          pltpu.VMEM((NBUF, STREAMS_PER_ITEM, ROWS_PER_STREAM), jnp.int32),
          pltpu.VMEM((NBUF, ITEM, n_classes), jnp.float32),
      ]
      + [pltpu.SemaphoreType.DMA] * (3 * NBUF),
  )
  def gather_kernel(table, idx3, out, idx_v, rows_v, *sems):
    idx_sem = sems[:NBUF]
    gat_sem = sems[NBUF:2 * NBUF]
    out_sem = sems[2 * NBUF:]
    wid = lax.axis_index("s") * NUM_CORES + lax.axis_index("c")
    first_item = wid * items_per_worker

    def item_of(g, b):
      return first_item + g * NBUF + b

    for b in range(NBUF):
      pltpu.async_copy(idx3.at[item_of(0, b)], idx_v.at[b], idx_sem[b])

    @pl.loop(0, n_groups)
    def _group(g):
      copies = [None] * NBUF
      for b in range(NBUF):
        m = item_of(g, b)
        pltpu.make_async_copy(idx3.at[m], idx_v.at[b], idx_sem[b]).wait()
        offset = (m // items_per_tree) * n_leaves
        for r in range(STREAMS_PER_ITEM):
          for v in range(ROWS_PER_STREAM // LANES):
            sl = pl.ds(v * LANES, LANES)
            idx_v[b, r, sl] = idx_v[b, r, sl] + offset
        @pl.when(g != 0)
        def _():
          pltpu.make_async_copy(
              rows_v.at[b], out.at[pl.ds(0, ITEM)], out_sem[b]).wait()
        copies[b] = [
            pltpu.async_copy(
                table.at[idx_v.at[b, r]],
                rows_v.at[b, pl.ds(r * ROWS_PER_STREAM, ROWS_PER_STREAM)],
                gat_sem[b])
            for r in range(STREAMS_PER_ITEM)
        ]

      for b in range(NBUF):
        m = item_of(g, b)
        for c in copies[b]:
          c.wait()
        pltpu.async_copy(rows_v.at[b], out.at[pl.ds(m * ITEM, ITEM)],
                         out_sem[b])

        @pl.when(g != n_groups - 1)
        def _():
          nxt = item_of(g + 1, b)
          pltpu.async_copy(idx3.at[nxt], idx_v.at[b], idx_sem[b])

    for b in range(NBUF):
      pltpu.make_async_copy(
          rows_v.at[b], out.at[pl.ds(0, ITEM)], out_sem[b]).wait()

  return gather_kernel


def kernel(leafs, idx):
  n_trees, n_leaves, n_classes = leafs.shape
  batch = idx.shape[1]
  table = leafs.reshape(n_trees * n_leaves, n_classes)
  idx3 = idx.astype(jnp.int32).reshape(-1, STREAMS_PER_ITEM, ROWS_PER_STREAM)
  out = _make_gather(n_trees, n_leaves, batch, n_classes)(table, idx3)
  return out.reshape(n_trees, batch, n_classes)


# speedup vs baseline: 1.0092x; 1.0013x over previous
import functools

import jax
import jax.numpy as jnp
from jax import lax
from jax.experimental import pallas as pl
from jax.experimental.pallas import tpu as pltpu
from jax.experimental.pallas import tpu_sc as plsc

NUM_CORES = 2
NUM_SUBCORES = 16
LANES = 16
NUM_WORKERS = NUM_CORES * NUM_SUBCORES

ROWS_PER_STREAM = 128
STREAMS_PER_ITEM = 4
ITEM = ROWS_PER_STREAM * STREAMS_PER_ITEM
NBUF = 2


def _make_gather(n_trees, n_leaves, batch, n_classes):
  total_rows = n_trees * batch
  n_items = total_rows // ITEM
  items_per_worker = n_items // NUM_WORKERS
  assert n_items == items_per_worker * NUM_WORKERS
  assert batch % ITEM == 0
  items_per_tree = batch // ITEM
  assert items_per_worker % NBUF == 0
  n_groups = items_per_worker // NBUF

  mesh = plsc.VectorSubcoreMesh(
      core_axis_name="c", subcore_axis_name="s",
      num_cores=NUM_CORES, num_subcores=NUM_SUBCORES)

  @functools.partial(
      pl.kernel,
      mesh=mesh,
      compiler_params=pltpu.CompilerParams(use_tc_tiling_on_sc=False),
      out_type=jax.ShapeDtypeStruct((n_trees, batch, n_classes), jnp.float32),
      scratch_types=[
          pltpu.VMEM((NBUF, STREAMS_PER_ITEM, ROWS_PER_STREAM), jnp.int32),
          pltpu.VMEM((NBUF, ITEM, n_classes), jnp.float32),
      ]
      + [pltpu.SemaphoreType.DMA] * (3 * NBUF),
  )
  def gather_kernel(table, idx4, out, idx_v, rows_v, *sems):
    idx_sem = sems[:NBUF]
    gat_sem = sems[NBUF:2 * NBUF]
    out_sem = sems[2 * NBUF:]
    wid = lax.axis_index("s") * NUM_CORES + lax.axis_index("c")
    first_item = wid * items_per_worker

    def item_of(g, b):
      return first_item + g * NBUF + b

    for b in range(NBUF):
      pltpu.async_copy(idx4.at[item_of(0, b)], idx_v.at[b], idx_sem[b])

    @pl.loop(0, n_groups)
    def _group(g):
      copies = [None] * NBUF
      for b in range(NBUF):
        m = item_of(g, b)
        tree = m // items_per_tree
        pltpu.make_async_copy(idx4.at[m], idx_v.at[b], idx_sem[b]).wait()
        @pl.when(g != 0)
        def _():
          pltpu.make_async_copy(
              rows_v.at[b], out.at[0, pl.ds(0, ITEM)], out_sem[b]).wait()
        copies[b] = [
            pltpu.async_copy(
                table.at[tree].at[idx_v.at[b, r]],
                rows_v.at[b, pl.ds(r * ROWS_PER_STREAM, ROWS_PER_STREAM)],
                gat_sem[b])
            for r in range(STREAMS_PER_ITEM)
        ]

      for b in range(NBUF):
        m = item_of(g, b)
        tree = m // items_per_tree
        boff = (m % items_per_tree) * ITEM
        for c in copies[b]:
          c.wait()
        pltpu.async_copy(rows_v.at[b], out.at[tree, pl.ds(boff, ITEM)],
                         out_sem[b])

        @pl.when(g != n_groups - 1)
        def _():
          nxt = item_of(g + 1, b)
          pltpu.async_copy(idx4.at[nxt], idx_v.at[b], idx_sem[b])

    for b in range(NBUF):
      pltpu.make_async_copy(
          rows_v.at[b], out.at[0, pl.ds(0, ITEM)], out_sem[b]).wait()

  return gather_kernel


def kernel(leafs, idx):
  n_trees, n_leaves, n_classes = leafs.shape
  batch = idx.shape[1]
  idx4 = idx.astype(jnp.int32).reshape(-1, STREAMS_PER_ITEM, ROWS_PER_STREAM)
  return _make_gather(n_trees, n_leaves, batch, n_classes)(leafs, idx4)


# speedup vs baseline: 5.2972x; 5.2488x over previous
import functools

import jax
import jax.numpy as jnp
from jax import lax
from jax.experimental import pallas as pl
from jax.experimental.pallas import tpu as pltpu
from jax.experimental.pallas import tpu_sc as plsc

NUM_CORES = 2
NUM_SUBCORES = 16
LANES = 16
NUM_WORKERS = NUM_CORES * NUM_SUBCORES

CHUNK = 8192


def _make_gather(n_trees, n_leaves, batch, n_classes):
  n_pairs = n_trees * n_classes
  pairs_per_worker = n_pairs // NUM_WORKERS
  assert n_pairs == pairs_per_worker * NUM_WORKERS
  assert batch % CHUNK == 0
  n_chunks = batch // CHUNK

  mesh = plsc.VectorSubcoreMesh(
      core_axis_name="c", subcore_axis_name="s",
      num_cores=NUM_CORES, num_subcores=NUM_SUBCORES)

  @functools.partial(
      pl.kernel,
      mesh=mesh,
      compiler_params=pltpu.CompilerParams(
          use_tc_tiling_on_sc=True, needs_layout_passes=False),
      out_type=jax.ShapeDtypeStruct((n_trees, n_classes, batch), jnp.float32),
      scratch_types=[
          pltpu.VMEM((n_leaves,), jnp.float32),
          pltpu.VMEM((CHUNK,), jnp.int32),
          pltpu.VMEM((CHUNK,), jnp.float32),
      ],
  )
  def gather_kernel(table, idx3, out, row_v, idx_v, out_v):
    wid = lax.axis_index("s") * NUM_CORES + lax.axis_index("c")
    first_pair = wid * pairs_per_worker

    @pl.loop(0, pairs_per_worker)
    def _pair(p):
      pair = first_pair + p
      tree = pair // n_classes
      cls = pair % n_classes
      pltpu.sync_copy(table.at[tree, cls], row_v)
      for h in range(n_chunks):
        pltpu.sync_copy(idx3.at[tree, pl.ds(h * CHUNK, CHUNK)], idx_v)

        @pl.loop(0, CHUNK // LANES, unroll=8)
        def _vec(k):
          sl = pl.ds(k * LANES, LANES)
          out_v[sl] = plsc.load_gather(row_v, [idx_v[sl]])

        pltpu.sync_copy(out_v, out.at[tree, cls, pl.ds(h * CHUNK, CHUNK)])

  return gather_kernel


def kernel(leafs, idx):
  n_trees, n_leaves, n_classes = leafs.shape
  batch = idx.shape[1]
  table = jnp.swapaxes(leafs, 1, 2)
  idx3 = idx.astype(jnp.int32)
  out = _make_gather(n_trees, n_leaves, batch, n_classes)(table, idx3)
  return jnp.swapaxes(out, 1, 2)


# speedup vs baseline: 5.7218x; 1.0802x over previous
import functools

import jax
import jax.numpy as jnp
from jax import lax
from jax.experimental import pallas as pl
from jax.experimental.pallas import tpu as pltpu
from jax.experimental.pallas import tpu_sc as plsc

NUM_CORES = 2
NUM_SUBCORES = 16
LANES = 16
NUM_WORKERS = NUM_CORES * NUM_SUBCORES

IDX_HALF = 8192
OUT_CHUNK = 4096


def _make_gather(n_trees, n_leaves, batch, n_classes):
  n_pairs = n_trees * n_classes
  pairs_per_worker = n_pairs // NUM_WORKERS
  assert n_pairs == pairs_per_worker * NUM_WORKERS
  assert batch == 2 * IDX_HALF and IDX_HALF == 2 * OUT_CHUNK

  mesh = plsc.VectorSubcoreMesh(
      core_axis_name="c", subcore_axis_name="s",
      num_cores=NUM_CORES, num_subcores=NUM_SUBCORES)

  @functools.partial(
      pl.kernel,
      mesh=mesh,
      compiler_params=pltpu.CompilerParams(
          use_tc_tiling_on_sc=True, needs_layout_passes=False),
      out_type=jax.ShapeDtypeStruct((n_trees, n_classes, batch), jnp.float32),
      scratch_types=[
          pltpu.VMEM((n_leaves,), jnp.float32),
          pltpu.VMEM((2, IDX_HALF), jnp.int32),
          pltpu.VMEM((2, OUT_CHUNK), jnp.float32),
          pltpu.SemaphoreType.DMA,
      ]
      + [pltpu.SemaphoreType.DMA] * 2
      + [pltpu.SemaphoreType.DMA] * 2,
  )
  def gather_kernel(table, idx3, out, row_v, idx_v, out_v, row_sem,
                    idx_sem0, idx_sem1, out_sem0, out_sem1):
    idx_sem = (idx_sem0, idx_sem1)
    out_sem = (out_sem0, out_sem1)
    wid = lax.axis_index("s") * NUM_CORES + lax.axis_index("c")
    first_pair = wid * pairs_per_worker

    def tree_cls(pair):
      return pair // n_classes, pair % n_classes

    tree0, _ = tree_cls(first_pair)
    for i in range(2):
      pltpu.async_copy(idx3.at[tree0, pl.ds(i * IDX_HALF, IDX_HALF)],
                       idx_v.at[i], idx_sem[i])

    @pl.loop(0, pairs_per_worker)
    def _pair(p):
      pair = first_pair + p
      tree, cls = tree_cls(pair)
      pltpu.async_copy(table.at[tree, cls], row_v, row_sem).wait()

      for i in range(2):
        pltpu.make_async_copy(
            idx3.at[tree, pl.ds(i * IDX_HALF, IDX_HALF)],
            idx_v.at[i], idx_sem[i]).wait()
        for c in range(2):
          boff = i * IDX_HALF + c * OUT_CHUNK
          if i == 0:
            @pl.when(p != 0)
            def _():
              pltpu.make_async_copy(
                  out_v.at[c], out.at[tree, cls, pl.ds(0, OUT_CHUNK)],
                  out_sem[c]).wait()
          else:
            pltpu.make_async_copy(
                out_v.at[c], out.at[tree, cls, pl.ds(0, OUT_CHUNK)],
                out_sem[c]).wait()

          @pl.loop(0, OUT_CHUNK // LANES, unroll=8)
          def _vec(k):
            sl = pl.ds(k * LANES, LANES)
            isl = pl.ds(c * OUT_CHUNK + k * LANES, LANES)
            out_v[c, sl] = plsc.load_gather(row_v, [idx_v[i, isl]])

          pltpu.async_copy(out_v.at[c], out.at[tree, cls, pl.ds(boff, OUT_CHUNK)],
                           out_sem[c])

        @pl.when(p != pairs_per_worker - 1)
        def _():
          ntree, _ = tree_cls(pair + 1)
          pltpu.async_copy(idx3.at[ntree, pl.ds(i * IDX_HALF, IDX_HALF)],
                           idx_v.at[i], idx_sem[i])

    for c in range(2):
      pltpu.make_async_copy(
          out_v.at[c], out.at[0, 0, pl.ds(0, OUT_CHUNK)], out_sem[c]).wait()

  return gather_kernel


def kernel(leafs, idx):
  n_trees, n_leaves, n_classes = leafs.shape
  batch = idx.shape[1]
  table = jnp.swapaxes(leafs, 1, 2)
  idx3 = idx.astype(jnp.int32)
  out = _make_gather(n_trees, n_leaves, batch, n_classes)(table, idx3)
  return jnp.swapaxes(out, 1, 2)


# speedup vs baseline: 10.4095x; 1.8193x over previous
import functools

import jax
import jax.numpy as jnp
from jax import lax
from jax.experimental import pallas as pl
from jax.experimental.pallas import tpu as pltpu
from jax.experimental.pallas import tpu_sc as plsc

NUM_CORES = 2
NUM_SUBCORES = 16
LANES = 16
NUM_WORKERS = NUM_CORES * NUM_SUBCORES

IDX_HALF = 8192
OUT_CHUNK = 4096


def _make_gather(n_trees, n_leaves, batch, n_classes):
  n_pairs = n_trees * n_classes
  pairs_per_worker = n_pairs // NUM_WORKERS
  assert n_pairs == pairs_per_worker * NUM_WORKERS
  assert batch == 2 * IDX_HALF and IDX_HALF == 2 * OUT_CHUNK

  mesh = plsc.VectorSubcoreMesh(
      core_axis_name="c", subcore_axis_name="s",
      num_cores=NUM_CORES, num_subcores=NUM_SUBCORES)

  @functools.partial(
      pl.kernel,
      mesh=mesh,
      compiler_params=pltpu.CompilerParams(
          use_tc_tiling_on_sc=True, needs_layout_passes=False),
      out_type=jax.ShapeDtypeStruct((n_trees, n_classes, batch), jnp.float32),
      scratch_types=[
          pltpu.VMEM((n_leaves,), jnp.float32),
          pltpu.VMEM((2, IDX_HALF), jnp.int32),
          pltpu.VMEM((2, OUT_CHUNK), jnp.float32),
          pltpu.SemaphoreType.DMA,
      ]
      + [pltpu.SemaphoreType.DMA] * 2
      + [pltpu.SemaphoreType.DMA] * 2,
  )
  def gather_kernel(table, idx3, out, row_v, idx_v, out_v, row_sem,
                    idx_sem0, idx_sem1, out_sem0, out_sem1):
    idx_sem = (idx_sem0, idx_sem1)
    out_sem = (out_sem0, out_sem1)
    wid = lax.axis_index("s") * NUM_CORES + lax.axis_index("c")
    first_pair = wid * pairs_per_worker

    def tree_cls(pair):
      return pair // n_classes, pair % n_classes

    tree0, _ = tree_cls(first_pair)
    for i in range(2):
      pltpu.async_copy(idx3.at[tree0, pl.ds(i * IDX_HALF, IDX_HALF)],
                       idx_v.at[i], idx_sem[i])

    @pl.loop(0, pairs_per_worker)
    def _pair(p):
      pair = first_pair + p
      tree, cls = tree_cls(pair)
      pltpu.async_copy(table.at[tree, cls], row_v, row_sem).wait()

      for i in range(2):
        pltpu.make_async_copy(
            idx3.at[tree, pl.ds(i * IDX_HALF, IDX_HALF)],
            idx_v.at[i], idx_sem[i]).wait()
        for c in range(2):
          boff = i * IDX_HALF + c * OUT_CHUNK
          if i == 0:
            @pl.when(p != 0)
            def _():
              pltpu.make_async_copy(
                  out_v.at[c], out.at[tree, cls, pl.ds(0, OUT_CHUNK)],
                  out_sem[c]).wait()
          else:
            pltpu.make_async_copy(
                out_v.at[c], out.at[tree, cls, pl.ds(0, OUT_CHUNK)],
                out_sem[c]).wait()

          @plsc.parallel_loop(0, OUT_CHUNK // LANES, unroll=8)
          def _vec(k):
            sl = pl.ds(k * LANES, LANES)
            isl = pl.ds(c * OUT_CHUNK + k * LANES, LANES)
            out_v[c, sl] = plsc.load_gather(row_v, [idx_v[i, isl]])

          pltpu.async_copy(out_v.at[c], out.at[tree, cls, pl.ds(boff, OUT_CHUNK)],
                           out_sem[c])

        @pl.when(p != pairs_per_worker - 1)
        def _():
          ntree, _ = tree_cls(pair + 1)
          pltpu.async_copy(idx3.at[ntree, pl.ds(i * IDX_HALF, IDX_HALF)],
                           idx_v.at[i], idx_sem[i])

    for c in range(2):
      pltpu.make_async_copy(
          out_v.at[c], out.at[0, 0, pl.ds(0, OUT_CHUNK)], out_sem[c]).wait()

  return gather_kernel


def kernel(leafs, idx):
  n_trees, n_leaves, n_classes = leafs.shape
  batch = idx.shape[1]
  table = jnp.swapaxes(leafs, 1, 2)
  idx3 = idx.astype(jnp.int32)
  out = _make_gather(n_trees, n_leaves, batch, n_classes)(table, idx3)
  return jnp.swapaxes(out, 1, 2)


# speedup vs baseline: 11.1558x; 1.0717x over previous
import functools

import jax
import jax.numpy as jnp
from jax import lax
from jax.experimental import pallas as pl
from jax.experimental.pallas import tpu as pltpu
from jax.experimental.pallas import tpu_sc as plsc

NUM_CORES = 2
NUM_SUBCORES = 16
LANES = 16
NUM_WORKERS = NUM_CORES * NUM_SUBCORES

IDX_HALF = 8192
OUT_CHUNK = 4096


def _make_gather(n_trees, n_leaves, batch, n_classes):
  n_pairs = n_trees * n_classes
  pairs_per_worker = n_pairs // NUM_WORKERS
  assert n_pairs == pairs_per_worker * NUM_WORKERS
  assert batch == 2 * IDX_HALF and IDX_HALF == 2 * OUT_CHUNK

  mesh = plsc.VectorSubcoreMesh(
      core_axis_name="c", subcore_axis_name="s",
      num_cores=NUM_CORES, num_subcores=NUM_SUBCORES)

  @functools.partial(
      pl.kernel,
      mesh=mesh,
      compiler_params=pltpu.CompilerParams(
          use_tc_tiling_on_sc=True, needs_layout_passes=False),
      out_type=jax.ShapeDtypeStruct((n_trees, n_classes, batch), jnp.float32),
      scratch_types=[
          pltpu.VMEM((n_leaves,), jnp.float32),
          pltpu.VMEM((2, IDX_HALF), jnp.int32),
          pltpu.VMEM((2, OUT_CHUNK), jnp.float32),
          pltpu.SemaphoreType.DMA,
      ]
      + [pltpu.SemaphoreType.DMA] * 2
      + [pltpu.SemaphoreType.DMA] * 2,
  )
  def gather_kernel(table, idx3, out, row_v, idx_v, out_v, row_sem,
                    idx_sem0, idx_sem1, out_sem0, out_sem1):
    idx_sem = (idx_sem0, idx_sem1)
    out_sem = (out_sem0, out_sem1)
    wid = lax.axis_index("s") * NUM_CORES + lax.axis_index("c")
    first_pair = wid * pairs_per_worker

    def tree_cls(pair):
      return pair // n_classes, pair % n_classes

    tree0, _ = tree_cls(first_pair)
    for i in range(2):
      pltpu.async_copy(idx3.at[tree0, pl.ds(i * IDX_HALF, IDX_HALF)],
                       idx_v.at[i], idx_sem[i])

    @pl.loop(0, pairs_per_worker)
    def _pair(p):
      pair = first_pair + p
      tree, cls = tree_cls(pair)
      pltpu.async_copy(table.at[tree, cls], row_v, row_sem).wait()

      reloaded = jnp.logical_or(p == 0, cls == 0)
      for i in range(2):
        @pl.when(reloaded)
        def _():
          pltpu.make_async_copy(
              idx3.at[tree, pl.ds(i * IDX_HALF, IDX_HALF)],
              idx_v.at[i], idx_sem[i]).wait()
        for c in range(2):
          boff = i * IDX_HALF + c * OUT_CHUNK
          if i == 0:
            @pl.when(p != 0)
            def _():
              pltpu.make_async_copy(
                  out_v.at[c], out.at[tree, cls, pl.ds(0, OUT_CHUNK)],
                  out_sem[c]).wait()
          else:
            pltpu.make_async_copy(
                out_v.at[c], out.at[tree, cls, pl.ds(0, OUT_CHUNK)],
                out_sem[c]).wait()

          @plsc.parallel_loop(0, OUT_CHUNK // LANES, unroll=8)
          def _vec(k):
            sl = pl.ds(k * LANES, LANES)
            isl = pl.ds(c * OUT_CHUNK + k * LANES, LANES)
            out_v[c, sl] = plsc.load_gather(row_v, [idx_v[i, isl]])

          pltpu.async_copy(out_v.at[c], out.at[tree, cls, pl.ds(boff, OUT_CHUNK)],
                           out_sem[c])

        @pl.when(jnp.logical_and(p != pairs_per_worker - 1,
                                 cls == n_classes - 1))
        def _():
          ntree, _ = tree_cls(pair + 1)
          pltpu.async_copy(idx3.at[ntree, pl.ds(i * IDX_HALF, IDX_HALF)],
                           idx_v.at[i], idx_sem[i])

    for c in range(2):
      pltpu.make_async_copy(
          out_v.at[c], out.at[0, 0, pl.ds(0, OUT_CHUNK)], out_sem[c]).wait()

  return gather_kernel


def kernel(leafs, idx):
  n_trees, n_leaves, n_classes = leafs.shape
  batch = idx.shape[1]
  table = jnp.swapaxes(leafs, 1, 2)
  idx3 = idx.astype(jnp.int32)
  out = _make_gather(n_trees, n_leaves, batch, n_classes)(table, idx3)
  return jnp.swapaxes(out, 1, 2)


# speedup vs baseline: 11.1595x; 1.0003x over previous
import functools

import jax
import jax.numpy as jnp
from jax import lax
from jax.experimental import pallas as pl
from jax.experimental.pallas import tpu as pltpu
from jax.experimental.pallas import tpu_sc as plsc

NUM_CORES = 2
NUM_SUBCORES = 16
LANES = 16
NUM_WORKERS = NUM_CORES * NUM_SUBCORES

IDX_HALF = 8192
OUT_CHUNK = 4096


def _make_gather(n_trees, n_leaves, batch, n_classes):
  n_pairs = n_trees * n_classes
  pairs_per_worker = n_pairs // NUM_WORKERS
  assert n_pairs == pairs_per_worker * NUM_WORKERS
  assert batch == 2 * IDX_HALF and IDX_HALF == 2 * OUT_CHUNK

  mesh = plsc.VectorSubcoreMesh(
      core_axis_name="c", subcore_axis_name="s",
      num_cores=NUM_CORES, num_subcores=NUM_SUBCORES)

  @functools.partial(
      pl.kernel,
      mesh=mesh,
      compiler_params=pltpu.CompilerParams(
          use_tc_tiling_on_sc=True, needs_layout_passes=False,
          skip_device_barrier=True),
      out_type=jax.ShapeDtypeStruct((n_trees, n_classes, batch), jnp.float32),
      scratch_types=[
          pltpu.VMEM((n_leaves,), jnp.float32),
          pltpu.VMEM((2, IDX_HALF), jnp.int32),
          pltpu.VMEM((2, OUT_CHUNK), jnp.float32),
          pltpu.SemaphoreType.DMA,
      ]
      + [pltpu.SemaphoreType.DMA] * 2
      + [pltpu.SemaphoreType.DMA] * 2,
  )
  def gather_kernel(table, idx3, out, row_v, idx_v, out_v, row_sem,
                    idx_sem0, idx_sem1, out_sem0, out_sem1):
    idx_sem = (idx_sem0, idx_sem1)
    out_sem = (out_sem0, out_sem1)
    wid = lax.axis_index("s") * NUM_CORES + lax.axis_index("c")
    first_pair = wid * pairs_per_worker

    def tree_cls(pair):
      return pair // n_classes, pair % n_classes

    tree0, _ = tree_cls(first_pair)
    for i in range(2):
      pltpu.async_copy(idx3.at[tree0, pl.ds(i * IDX_HALF, IDX_HALF)],
                       idx_v.at[i], idx_sem[i])

    @pl.loop(0, pairs_per_worker)
    def _pair(p):
      pair = first_pair + p
      tree, cls = tree_cls(pair)
      pltpu.async_copy(table.at[tree, cls], row_v, row_sem).wait()

      reloaded = jnp.logical_or(p == 0, cls == 0)
      for i in range(2):
        @pl.when(reloaded)
        def _():
          pltpu.make_async_copy(
              idx3.at[tree, pl.ds(i * IDX_HALF, IDX_HALF)],
              idx_v.at[i], idx_sem[i]).wait()
        for c in range(2):
          boff = i * IDX_HALF + c * OUT_CHUNK
          if i == 0:
            @pl.when(p != 0)
            def _():
              pltpu.make_async_copy(
                  out_v.at[c], out.at[tree, cls, pl.ds(0, OUT_CHUNK)],
                  out_sem[c]).wait()
          else:
            pltpu.make_async_copy(
                out_v.at[c], out.at[tree, cls, pl.ds(0, OUT_CHUNK)],
                out_sem[c]).wait()

          @plsc.parallel_loop(0, OUT_CHUNK // LANES, unroll=8)
          def _vec(k):
            sl = pl.ds(k * LANES, LANES)
            isl = pl.ds(c * OUT_CHUNK + k * LANES, LANES)
            out_v[c, sl] = plsc.load_gather(row_v, [idx_v[i, isl]])

          pltpu.async_copy(out_v.at[c], out.at[tree, cls, pl.ds(boff, OUT_CHUNK)],
                           out_sem[c])

        @pl.when(jnp.logical_and(p != pairs_per_worker - 1,
                                 cls == n_classes - 1))
        def _():
          ntree, _ = tree_cls(pair + 1)
          pltpu.async_copy(idx3.at[ntree, pl.ds(i * IDX_HALF, IDX_HALF)],
                           idx_v.at[i], idx_sem[i])

    for c in range(2):
      pltpu.make_async_copy(
          out_v.at[c], out.at[0, 0, pl.ds(0, OUT_CHUNK)], out_sem[c]).wait()

  return gather_kernel


def kernel(leafs, idx):
  n_trees, n_leaves, n_classes = leafs.shape
  batch = idx.shape[1]
  table = jnp.swapaxes(leafs, 1, 2)
  idx3 = idx.astype(jnp.int32)
  out = _make_gather(n_trees, n_leaves, batch, n_classes)(table, idx3)
  return jnp.swapaxes(out, 1, 2)
